# Initial kernel scaffold; baseline (speedup 1.0000x reference)
import jax
import jax.numpy as jnp
from jax.experimental import pallas as pl


def kernel(x, token_table, pos_table):
    raise NotImplementedError("write your pallas kernel here")


# speedup vs baseline: 5.3776x; 5.3776x over previous
import functools

import jax
import jax.numpy as jnp
from jax import lax
from jax.experimental import pallas as pl
from jax.experimental.pallas import tpu as pltpu
from jax.experimental.pallas import tpu_sc as plsc

NC, NS = 2, 16
NW = NC * NS
CHUNK = 128
LANES = 16


def _sc_embed(x_flat, token_table, pos_table, *, n, embed, maxlen):
    per_w = n // NW
    n_chunks = per_w // CHUNK
    mesh = plsc.VectorSubcoreMesh(core_axis_name="c", subcore_axis_name="s")

    @functools.partial(
        pl.kernel,
        out_type=jax.ShapeDtypeStruct((n, embed), jnp.float32),
        mesh=mesh,
        scratch_types=[
            pltpu.VMEM((CHUNK,), jnp.int32),
            pltpu.VMEM((CHUNK,), jnp.int32),
            pltpu.VMEM((CHUNK, embed), jnp.float32),
            pltpu.VMEM((CHUNK, embed), jnp.float32),
            pltpu.SemaphoreType.DMA,
            pltpu.SemaphoreType.DMA,
        ],
        compiler_params=pltpu.CompilerParams(use_tc_tiling_on_sc=False),
    )
    def sc_kernel(x_hbm, tok_hbm, pos_hbm, out_hbm,
                  idx_v, pidx_v, tok_v, pos_v, sem_t, sem_p):
        wid = lax.axis_index("s") * NC + lax.axis_index("c")
        wbase = wid * per_w

        @pl.loop(0, n_chunks)
        def _(c):
            base = wbase + c * CHUNK
            pltpu.sync_copy(x_hbm.at[pl.ds(base, CHUNK)], idx_v)
            for j in range(CHUNK // LANES):
                xg = idx_v[pl.ds(j * LANES, LANES)]
                s = lax.rem(base + j * LANES, maxlen)
                lvec = lax.rem(s + lax.iota(jnp.int32, LANES), maxlen)
                pidx_v[pl.ds(j * LANES, LANES)] = (
                    (lvec + 1) * jnp.minimum(xg, 1))
            cp_t = pltpu.async_copy(tok_hbm.at[idx_v], tok_v, sem_t)
            cp_p = pltpu.async_copy(pos_hbm.at[pidx_v], pos_v, sem_p)
            cp_t.wait()
            cp_p.wait()

            @pl.loop(0, CHUNK, step=8)
            def _(r0):
                for dr in range(8):
                    for h in range(embed // LANES):
                        sl = (r0 + dr, pl.ds(h * LANES, LANES))
                        plsc.addupdate(tok_v.at[sl], pos_v[sl])

            pltpu.sync_copy(tok_v, out_hbm.at[pl.ds(base, CHUNK)])

    return sc_kernel(x_flat, token_table, pos_table)


def kernel(x, token_table, pos_table):
    b, maxlen = x.shape
    embed = token_table.shape[1]
    n = b * maxlen
    x_flat = x.reshape(n).astype(jnp.int32)
    out = _sc_embed(x_flat, token_table, pos_table,
                    n=n, embed=embed, maxlen=maxlen)
    return out.reshape(b, maxlen, embed)


# speedup vs baseline: 6.0555x; 1.1261x over previous
import functools

import jax
import jax.numpy as jnp
from jax import lax
from jax.experimental import pallas as pl
from jax.experimental.pallas import tpu as pltpu
from jax.experimental.pallas import tpu_sc as plsc

NC, NS = 2, 16
NW = NC * NS
GW = 128
NG = 4
CHUNK = GW * NG
LANES = 16


def _sc_embed(x_rows, token_table, pos_table, *, n, embed, maxlen):
    per_w = n // NW
    n_chunks = per_w // CHUNK
    lcm = (CHUNK * maxlen) // 8
    period = lcm // CHUNK
    mesh = plsc.VectorSubcoreMesh(core_axis_name="c", subcore_axis_name="s")

    @functools.partial(
        pl.kernel,
        out_type=jax.ShapeDtypeStruct((n, embed), jnp.float32),
        mesh=mesh,
        scratch_types=[
            pltpu.VMEM((2, NG, GW), jnp.int32),
            pltpu.VMEM((2, NG, GW), jnp.int32),
            pltpu.VMEM((2, CHUNK, embed), jnp.float32),
            pltpu.VMEM((2, CHUNK, embed), jnp.float32),
            pltpu.VMEM((lcm,), jnp.int32),
            pltpu.SemaphoreType.DMA,
            pltpu.SemaphoreType.DMA,
            pltpu.SemaphoreType.DMA,
            pltpu.SemaphoreType.DMA,
            pltpu.SemaphoreType.DMA,
            pltpu.SemaphoreType.DMA,
        ],
        compiler_params=pltpu.CompilerParams(use_tc_tiling_on_sc=False),
    )
    def sc_kernel(x_hbm, tok_hbm, pos_hbm, out_hbm,
                  idx_v, pidx_v, tok_v, pos_v, lp1_v,
                  si0, si1, sg0, sg1, so0, so1):
        wid = lax.axis_index("s") * NC + lax.axis_index("c")
        chunk0 = wid * n_chunks

        @pl.loop(0, lcm // LANES)
        def _(i):
            lp1_v[pl.ds(i * LANES, LANES)] = (
                lax.rem(i * LANES + lax.iota(jnp.int32, LANES), maxlen) + 1)

        def fire_idx(slot, c, sem):
            row = (chunk0 + c) * NG
            pltpu.async_copy(x_hbm.at[pl.ds(row, NG)], idx_v.at[slot], sem)

        def wait_idx(slot, sem):
            pltpu.make_async_copy(
                x_hbm.at[pl.ds(0, NG)], idx_v.at[slot], sem).wait()

        def wait_out(slot, sem):
            pltpu.make_async_copy(
                tok_v.at[slot], out_hbm.at[pl.ds(0, CHUNK)], sem).wait()

        def phase_fire(slot, c, sem_i, sem_o):
            wait_idx(slot, sem_i)
            om = lax.rem(c, period) * CHUNK
            for k in range(NG):
                for j in range(GW // LANES):
                    xg = idx_v[slot, k, pl.ds(j * LANES, LANES)]
                    lp = lp1_v[pl.ds(om + k * GW + j * LANES, LANES)]
                    pidx_v[slot, k, pl.ds(j * LANES, LANES)] = (
                        lp * jnp.minimum(xg, 1))
            @pl.when(c >= 2)
            def _():
                wait_out(slot, sem_o)

        def fire_gathers(slot, sem_g):
            for k in range(NG):
                pltpu.async_copy(tok_hbm.at[idx_v.at[slot, k]],
                                 tok_v.at[slot, pl.ds(k * GW, GW)], sem_g)
                pltpu.async_copy(pos_hbm.at[pidx_v.at[slot, k]],
                                 pos_v.at[slot, pl.ds(k * GW, GW)], sem_g)

        def phase_drain(slot, c, sem_i, sem_g, sem_o):
            for k in range(NG):
                pltpu.make_async_copy(
                    tok_hbm.at[pl.ds(0, GW)],
                    tok_v.at[slot, pl.ds(k * GW, GW)], sem_g).wait()
                pltpu.make_async_copy(
                    pos_hbm.at[pl.ds(0, GW)],
                    pos_v.at[slot, pl.ds(k * GW, GW)], sem_g).wait()

            @pl.when(c + 2 < n_chunks)
            def _():
                fire_idx(slot, c + 2, sem_i)

            @pl.loop(0, CHUNK, step=8)
            def _(r0):
                for dr in range(8):
                    for h in range(embed // LANES):
                        sl = (r0 + dr, pl.ds(h * LANES, LANES))
                        plsc.addupdate(tok_v.at[slot].at[sl],
                                       pos_v[(slot,) + sl])

            pltpu.async_copy(
                tok_v.at[slot],
                out_hbm.at[pl.ds((chunk0 + c) * CHUNK, CHUNK)], sem_o)

        fire_idx(0, 0, si0)
        fire_idx(1, 1, si1)

        @pl.loop(0, n_chunks, step=2)
        def _(c0):
            phase_fire(0, c0, si0, so0)
            fire_gathers(0, sg0)
            phase_fire(1, c0 + 1, si1, so1)
            fire_gathers(1, sg1)
            phase_drain(0, c0, si0, sg0, so0)
            phase_drain(1, c0 + 1, si1, sg1, so1)

        wait_out(0, so0)
        wait_out(1, so1)

    return sc_kernel(x_rows, token_table, pos_table)


def kernel(x, token_table, pos_table):
    b, maxlen = x.shape
    embed = token_table.shape[1]
    n = b * maxlen
    x_rows = x.reshape(n // GW, GW).astype(jnp.int32)
    out = _sc_embed(x_rows, token_table, pos_table,
                    n=n, embed=embed, maxlen=maxlen)
    return out.reshape(b, maxlen, embed)
